# Initial kernel scaffold; baseline (speedup 1.0000x reference)
import jax
import jax.numpy as jnp
from jax.experimental import pallas as pl


def kernel(input_feat, pos_table, ln_gamma, ln_beta):
    raise NotImplementedError("write your pallas kernel here")


# speedup vs baseline: 4.4561x; 4.4561x over previous
import functools

import jax
import jax.numpy as jnp
from jax.experimental import pallas as pl


def _ln_body(inp_ref, pos_ref, gamma_ref, beta_ref, out_ref):
    x = inp_ref[...] + pos_ref[...][None]
    mean = jnp.mean(x, axis=-1, keepdims=True)
    xc = x - mean
    var = jnp.mean(xc * xc, axis=-1, keepdims=True)
    inv = jax.lax.rsqrt(var + 1e-5)
    out_ref[...] = xc * inv * gamma_ref[...] + beta_ref[...]


def kernel(input_feat, pos_table, ln_gamma, ln_beta):
    B, S, H = input_feat.shape
    R = 256
    grid = (S // R,)
    gamma2 = ln_gamma.reshape(1, H)
    beta2 = ln_beta.reshape(1, H)
    out = pl.pallas_call(
        _ln_body,
        grid=grid,
        in_specs=[
            pl.BlockSpec((B, R, H), lambda j: (0, j, 0)),
            pl.BlockSpec((R, H), lambda j: (j, 0)),
            pl.BlockSpec((1, H), lambda j: (0, 0)),
            pl.BlockSpec((1, H), lambda j: (0, 0)),
        ],
        out_specs=pl.BlockSpec((B, R, H), lambda j: (0, j, 0)),
        out_shape=jax.ShapeDtypeStruct((B, S, H), jnp.float32),
    )(input_feat, pos_table, gamma2, beta2)
    return out
